# baseline (speedup 1.0000x reference)
import jax
import jax.numpy as jnp
from jax import lax
from jax.experimental import pallas as pl
from jax.experimental.pallas import tpu as pltpu
from jax.experimental.pallas import tpu_sc as plsc

N = 10000
E = 320000
E_PAD = 327680
N_INTER = 3
NF = 128
FH = NF // 2
NB = 25
CUTOFF = 5.0

NC = 2
NS = 16
NW = NC * NS
EPW = E_PAD // NW
EPT = E_PAD // NS
ROWS_PER_TILE = 632
NACC = 10112
N_PADR = 10240
ZROWS = 128

B_E = 2048
B_N = 2000

CE_D2 = 512
CE = 128

_INTERP = False


def _smear_body(d2_ref, fij_ref, cij_ref):
    d2 = d2_ref[...]
    r = jnp.sqrt(d2 + 1e-12)
    x = r * (jnp.pi / CUTOFF)
    t = x * x
    cs = (0.99999999, -0.499999918, 4.16665243e-02, -1.38879703e-03,
          2.47734208e-05, -2.71133377e-07, 1.73689959e-09)
    p = cs[6]
    for cc in cs[5::-1]:
        p = p * t + cc
    c = 0.5 * (p + 1.0)
    cij_ref[...] = jnp.where(r < CUTOFF, c, 0.0)
    width = CUTOFF / (NB - 1)
    coeff = -0.5 / (width * width)
    k = jax.lax.broadcasted_iota(jnp.int32, (1, 32), 1)
    off = k.astype(jnp.float32) * width
    diff = r - off
    fij = jnp.exp(coeff * diff * diff)
    fij = jnp.where(k < NB, fij, 0.0)
    fij_ref[...] = fij.astype(jnp.bfloat16)


def _smear(d2_col):
    grid = E_PAD // B_E
    return pl.pallas_call(
        _smear_body,
        grid=(grid,),
        in_specs=[pl.BlockSpec((B_E, 1), lambda i: (i, 0))],
        out_specs=[
            pl.BlockSpec((B_E, 32), lambda i: (i, 0)),
            pl.BlockSpec((B_E, 1), lambda i: (i, 0)),
        ],
        out_shape=[
            jax.ShapeDtypeStruct((E_PAD, 32), jnp.bfloat16),
            jax.ShapeDtypeStruct((E_PAD, 1), jnp.float32),
        ],
        interpret=_INTERP,
    )(d2_col)


def _edge_filter_body(fij_ref, cij_ref, wf1_ref, wf2h_ref, bf1_ref, bf2h_ref,
                      wc_ref):
    fij = fij_ref[...]
    u = jnp.dot(fij, wf1_ref[0], preferred_element_type=jnp.float32) + bf1_ref[0]
    su = (u * jax.nn.sigmoid(u)).astype(jnp.bfloat16)
    c = cij_ref[...]
    for half in range(2):
        w = jnp.dot(su, wf2h_ref[0, half], preferred_element_type=jnp.float32)
        wc_ref[0, half] = (w + bf2h_ref[0, half]) * c


def _edge_filter_all(fij, cij, wf1p, wf2h, bf1, bf2h):
    return pl.pallas_call(
        _edge_filter_body,
        grid=(N_INTER, E_PAD // B_E),
        in_specs=[
            pl.BlockSpec((B_E, 32), lambda i, j: (j, 0)),
            pl.BlockSpec((B_E, 1), lambda i, j: (j, 0)),
            pl.BlockSpec((1, 32, NF), lambda i, j: (i, 0, 0)),
            pl.BlockSpec((1, 2, NF, FH), lambda i, j: (i, 0, 0, 0)),
            pl.BlockSpec((1, 1, NF), lambda i, j: (i, 0, 0)),
            pl.BlockSpec((1, 2, 1, FH), lambda i, j: (i, 0, 0, 0)),
        ],
        out_specs=pl.BlockSpec((1, 2, B_E, FH), lambda i, j: (i, 0, j, 0)),
        out_shape=jax.ShapeDtypeStruct((N_INTER, 2, E_PAD, FH), jnp.float32),
        interpret=_INTERP,
    )(fij, cij, wf1p, wf2h, bf1, bf2h)


def _init_body(z_ref, emb_ref, winh_ref, feat_ref, h_ref):
    z = z_ref[...]
    lane = jax.lax.broadcasted_iota(jnp.int32, (1, NF), 1)
    oh = (z == lane).astype(jnp.float32)
    feat = jnp.dot(oh, emb_ref[...], preferred_element_type=jnp.float32)
    feat_ref[...] = feat
    for half in range(2):
        h_ref[half] = jnp.dot(feat, winh_ref[half],
                              preferred_element_type=jnp.float32)


def _init_feat(z_col, emb_pad, winh0):
    grid = N // B_N
    return pl.pallas_call(
        _init_body,
        grid=(grid,),
        in_specs=[
            pl.BlockSpec((B_N, 1), lambda i: (i, 0)),
            pl.BlockSpec((NF, NF), lambda i: (0, 0)),
            pl.BlockSpec((2, NF, FH), lambda i: (0, 0, 0)),
        ],
        out_specs=[
            pl.BlockSpec((B_N, NF), lambda i: (i, 0)),
            pl.BlockSpec((2, B_N, FH), lambda i: (0, i, 0)),
        ],
        out_shape=[
            jax.ShapeDtypeStruct((N, NF), jnp.float32),
            jax.ShapeDtypeStruct((2, N, FH), jnp.float32),
        ],
        interpret=_INTERP,
    )(z_col, emb_pad, winh0)


def _node_body(feat_ref, m_ref, wout1_ref, wout2_ref, bout1_ref, bout2_ref,
               gamma_ref, beta_ref, winh_next_ref, feat_out_ref, h_out_ref):
    m = jnp.concatenate([m_ref[0], m_ref[1]], axis=1)
    u = jnp.dot(m, wout1_ref[...], preferred_element_type=jnp.float32) + bout1_ref[...]
    su = u * jax.nn.sigmoid(u)
    mm = jnp.dot(su, wout2_ref[...], preferred_element_type=jnp.float32) + bout2_ref[...]
    f = feat_ref[...] + mm
    mu = jnp.mean(f, axis=1, keepdims=True)
    d = f - mu
    var = jnp.mean(d * d, axis=1, keepdims=True)
    fn = gamma_ref[...] * d * jax.lax.rsqrt(var + 1e-5) + beta_ref[...]
    feat_out_ref[...] = fn
    for half in range(2):
        h_out_ref[half] = jnp.dot(fn, winh_next_ref[half],
                                  preferred_element_type=jnp.float32)


def _node_update(feat, m2, wout1, wout2, bout1, bout2, gamma, beta, winh_next):
    grid = N // B_N
    return pl.pallas_call(
        _node_body,
        grid=(grid,),
        in_specs=[
            pl.BlockSpec((B_N, NF), lambda i: (i, 0)),
            pl.BlockSpec((2, B_N, FH), lambda i: (0, i, 0)),
            pl.BlockSpec((NF, NF), lambda i: (0, 0)),
            pl.BlockSpec((NF, NF), lambda i: (0, 0)),
            pl.BlockSpec((1, NF), lambda i: (0, 0)),
            pl.BlockSpec((1, NF), lambda i: (0, 0)),
            pl.BlockSpec((1, NF), lambda i: (0, 0)),
            pl.BlockSpec((1, NF), lambda i: (0, 0)),
            pl.BlockSpec((2, NF, FH), lambda i: (0, 0, 0)),
        ],
        out_specs=[
            pl.BlockSpec((B_N, NF), lambda i: (i, 0)),
            pl.BlockSpec((2, B_N, FH), lambda i: (0, i, 0)),
        ],
        out_shape=[
            jax.ShapeDtypeStruct((N, NF), jnp.float32),
            jax.ShapeDtypeStruct((2, N, FH), jnp.float32),
        ],
        interpret=_INTERP,
    )(feat, m2, wout1, wout2, bout1, bout2, gamma, beta, winh_next)


def _sc_mesh():
    return plsc.VectorSubcoreMesh(core_axis_name="c", subcore_axis_name="s",
                                  num_cores=NC, num_subcores=NS)


def _sc_d2_body(px_hbm, py_hbm, pz_hbm, src_hbm, dst_hbm, d2_hbm,
                px_v, py_v, pz_v, src_v, dst_v, d2_v):
    cid = lax.axis_index("c")
    sid = lax.axis_index("s")
    wid = sid * NC + cid
    pltpu.sync_copy(px_hbm, px_v)
    pltpu.sync_copy(py_hbm, py_v)
    pltpu.sync_copy(pz_hbm, pz_v)
    lanes = jnp.arange(16, dtype=jnp.int32)
    wbase = wid * EPW

    @pl.loop(0, EPW // CE_D2)
    def _chunk(ci):
        base = wbase + ci * CE_D2
        pltpu.sync_copy(src_hbm.at[pl.ds(base, CE_D2)], src_v)
        pltpu.sync_copy(dst_hbm.at[pl.ds(base, CE_D2)], dst_v)

        @pl.loop(0, CE_D2 // 16)
        def _vec(j):
            sl = pl.ds(j * 16, 16)
            si = src_v[sl]
            di = dst_v[sl]
            dx = plsc.load_gather(px_v, [di]) - plsc.load_gather(px_v, [si])
            dy = plsc.load_gather(py_v, [di]) - plsc.load_gather(py_v, [si])
            dz = plsc.load_gather(pz_v, [di]) - plsc.load_gather(pz_v, [si])
            d2 = dx * dx + dy * dy + dz * dz
            eid = base + j * 16 + lanes
            d2_v[sl] = jnp.where(eid < E, d2, 100.0)

        pltpu.sync_copy(d2_v, d2_hbm.at[pl.ds(base, CE_D2)])


def _sc_d2(px, py, pz, src, dst):
    f = pl.kernel(
        _sc_d2_body,
        out_type=jax.ShapeDtypeStruct((E_PAD,), jnp.float32),
        mesh=_sc_mesh(),
        compiler_params=pltpu.CompilerParams(needs_layout_passes=False),
        scratch_types=[
            pltpu.VMEM((N_PADR,), jnp.float32),
            pltpu.VMEM((N_PADR,), jnp.float32),
            pltpu.VMEM((N_PADR,), jnp.float32),
            pltpu.VMEM((CE_D2,), jnp.int32),
            pltpu.VMEM((CE_D2,), jnp.int32),
            pltpu.VMEM((CE_D2,), jnp.float32),
        ],
    )
    return f(px, py, pz, src, dst)


def _sc_cfconv_body(h_hbm, wc_hbm, src_hbm, dst_hbm, out_hbm,
                    src_blk, dst_blk, hrows0, hrows1, wcv0, wcv1, sbuf0, sbuf1,
                    macc, gsem0, gsem1, wsem0, wsem1, ssem0, ssem1):
    cid = lax.axis_index("c")
    sid = lax.axis_index("s")
    hrows = (hrows0, hrows1)
    wcv = (wcv0, wcv1)
    sbuf = (sbuf0, sbuf1)
    gsem = (gsem0, gsem1)
    wsem = (wsem0, wsem1)
    ssem = (ssem0, ssem1)
    NCH = EPT // CE

    @pl.loop(0, ZROWS)
    def _zr(i):
        for k2 in range(FH // 16):
            hrows0[i, pl.ds(k2 * 16, 16)] = jnp.zeros((16,), jnp.float32)

    row0 = sid * ROWS_PER_TILE
    for k in range(ROWS_PER_TILE // ZROWS):
        pltpu.sync_copy(hrows0.at[pl.ds(0, ZROWS)],
                        macc.at[pl.ds(row0 + k * ZROWS, ZROWS)])
    ztail = ROWS_PER_TILE % ZROWS
    if ztail:
        pltpu.sync_copy(hrows0.at[pl.ds(0, ztail)],
                        macc.at[pl.ds(row0 + ROWS_PER_TILE - ztail, ztail)])
    plsc.subcore_barrier()

    pltpu.sync_copy(src_hbm.at[pl.ds(sid * NCH, NCH)], src_blk)
    pltpu.sync_copy(dst_hbm.at[pl.ds(sid * NCH, NCH)], dst_blk)

    def start_gather(ci, b):
        pltpu.async_copy(h_hbm.at[cid].at[src_blk.at[ci]], hrows[b], gsem[b])

    def start_wc(ci, b):
        base = sid * EPT + ci * CE
        pltpu.async_copy(wc_hbm.at[cid, pl.ds(base, CE)], wcv[b], wsem[b])

    start_gather(0, 0)
    start_gather(1, 1)
    start_wc(0, 0)
    start_wc(1, 1)

    @pl.loop(0, NCH, step=2)
    def _chunk(ci0):
        for b in range(2):
            ci = ci0 + b
            base = sid * EPT + ci * CE
            pltpu.make_async_copy(h_hbm.at[cid].at[src_blk.at[ci]], hrows[b],
                                  gsem[b]).wait()
            pltpu.make_async_copy(wc_hbm.at[cid, pl.ds(base, CE)], wcv[b],
                                  wsem[b]).wait()

            @pl.when(ci >= 2)
            def _():
                pltpu.make_async_copy(sbuf[b], macc.at[dst_blk.at[ci - 2]],
                                      ssem[b]).wait()

            @pl.loop(0, CE)
            def _row(i):
                for k2 in range(FH // 16):
                    sl = pl.ds(k2 * 16, 16)
                    sbuf[b][i, sl] = hrows[b][i, sl] * wcv[b][i, sl]

            pltpu.async_copy(sbuf[b], macc.at[dst_blk.at[ci]], ssem[b],
                             add=True)

            @pl.when(ci + 2 < NCH)
            def _():
                start_gather(ci + 2, b)
                start_wc(ci + 2, b)

    for b in range(2):
        pltpu.make_async_copy(sbuf[b], macc.at[dst_blk.at[NCH - 2 + b]],
                              ssem[b]).wait()

    plsc.subcore_barrier()
    pltpu.sync_copy(macc.at[pl.ds(row0, ROWS_PER_TILE)],
                    out_hbm.at[cid, pl.ds(row0, ROWS_PER_TILE)])


def _sc_cfconv(h2, wc2, src2, dst2):
    f = pl.kernel(
        _sc_cfconv_body,
        out_type=jax.ShapeDtypeStruct((NC, NACC, FH), jnp.float32),
        mesh=_sc_mesh(),
        compiler_params=pltpu.CompilerParams(needs_layout_passes=False,
                                             use_tc_tiling_on_sc=False),
        scratch_types=(
            [pltpu.VMEM((EPT // CE, CE), jnp.int32) for _ in range(2)]
            + [pltpu.VMEM((CE, FH), jnp.float32) for _ in range(6)]
            + [pltpu.VMEM_SHARED((NACC, FH), jnp.float32)]
            + [pltpu.SemaphoreType.DMA for _ in range(6)]
        ),
    )
    return f(h2, wc2, src2, dst2)


def kernel(z, pos, edge_index, emb, Wf1, bf1, Wf2, bf2, Win, Wout1, bout1, Wout2, bout2, gamma, beta):
    src = jnp.pad(edge_index[0].astype(jnp.int32), (0, E_PAD - E))
    dst = jnp.pad(edge_index[1].astype(jnp.int32), (0, E_PAD - E))

    pos_t = jnp.pad(pos.T, ((0, 0), (0, N_PADR - N)))
    d2 = _sc_d2(pos_t[0], pos_t[1], pos_t[2], src, dst)
    d2_col = d2.reshape(E_PAD, 1)

    wf1p = jnp.pad(Wf1, ((0, 0), (0, 32 - NB), (0, 0))).astype(jnp.bfloat16)
    emb_pad = jnp.pad(emb, ((0, NF - emb.shape[0]), (0, 0)))
    wf2h = Wf2.reshape(N_INTER, NF, 2, FH).transpose(0, 2, 1, 3).astype(jnp.bfloat16)
    winh = Win.reshape(N_INTER, NF, 2, FH).transpose(0, 2, 1, 3)
    bf2h = bf2.reshape(N_INTER, 2, 1, FH)
    b2 = lambda b: b.reshape(N_INTER, 1, NF)
    bf1c, bout1c, bout2c = b2(bf1), b2(bout1), b2(bout2)
    gammac, betac = b2(gamma), b2(beta)

    src2 = src.reshape(E_PAD // CE, CE)
    dst2 = dst.reshape(E_PAD // CE, CE)
    z_col = z.reshape(N, 1).astype(jnp.int32)

    fij, cij = _smear(d2_col)
    wcs = _edge_filter_all(fij, cij, wf1p, wf2h,
                           bf1c.reshape(N_INTER, 1, NF),
                           bf2h)
    feat, h2 = _init_feat(z_col, emb_pad, winh[0])

    for i in range(N_INTER):
        m2 = _sc_cfconv(h2, wcs[i], src2, dst2)
        feat, h2 = _node_update(feat, m2[:, :N], Wout1[i], Wout2[i],
                                bout1c[i], bout2c[i], gammac[i], betac[i],
                                winh[(i + 1) % N_INTER])
    return feat


# speedup vs baseline: 1.1062x; 1.1062x over previous
import jax
import jax.numpy as jnp
from jax import lax
from jax.experimental import pallas as pl
from jax.experimental.pallas import tpu as pltpu
from jax.experimental.pallas import tpu_sc as plsc

N = 10000
E = 320000
E_PAD = 327680
N_INTER = 3
NF = 128
FH = NF // 2
NB = 25
CUTOFF = 5.0

NC = 2
NS = 16
NW = NC * NS
EPW = E_PAD // NW
EPT = E_PAD // NS
ROWS_PER_TILE = 632
NACC = 10112
N_PADR = 10240
ZROWS = 128

B_E = 1024
B_N = 2000

CE_D2 = 512
CE = 128

_INTERP = False


def _smear_body(d2_ref, fij_ref, cij_ref):
    d2 = d2_ref[...]
    r = jnp.sqrt(d2 + 1e-12)
    x = r * (jnp.pi / CUTOFF)
    t = x * x
    cs = (0.99999999, -0.499999918, 4.16665243e-02, -1.38879703e-03,
          2.47734208e-05, -2.71133377e-07, 1.73689959e-09)
    p = cs[6]
    for cc in cs[5::-1]:
        p = p * t + cc
    c = 0.5 * (p + 1.0)
    cij_ref[...] = jnp.where(r < CUTOFF, c, 0.0)
    width = CUTOFF / (NB - 1)
    coeff = -0.5 / (width * width)
    k = jax.lax.broadcasted_iota(jnp.int32, (1, 32), 1)
    off = k.astype(jnp.float32) * width
    diff = r - off
    fij = jnp.exp(coeff * diff * diff)
    fij = jnp.where(k < NB, fij, 0.0)
    fij_ref[...] = fij.astype(jnp.bfloat16)


def _smear(d2_col):
    grid = E_PAD // B_E
    return pl.pallas_call(
        _smear_body,
        grid=(grid,),
        in_specs=[pl.BlockSpec((B_E, 1), lambda i: (i, 0))],
        out_specs=[
            pl.BlockSpec((B_E, 32), lambda i: (i, 0)),
            pl.BlockSpec((B_E, 1), lambda i: (i, 0)),
        ],
        out_shape=[
            jax.ShapeDtypeStruct((E_PAD, 32), jnp.bfloat16),
            jax.ShapeDtypeStruct((E_PAD, 1), jnp.float32),
        ],
        interpret=_INTERP,
    )(d2_col)


def _edge_filter_body(fij_ref, cij_ref, wf1_ref, wf2h_ref, bf1_ref, bf2h_ref,
                      wc_ref):
    fij = fij_ref[...]
    u = jnp.dot(fij, wf1_ref[...], preferred_element_type=jnp.float32) + bf1_ref[...]
    su = (u * jax.nn.sigmoid(u)).astype(jnp.bfloat16)
    c = cij_ref[...]
    for half in range(2):
        w = jnp.dot(su, wf2h_ref[half], preferred_element_type=jnp.float32)
        wc_ref[half] = (w + bf2h_ref[half]) * c


def _edge_filter(fij, cij, wf1p, wf2h, bf1, bf2h):
    grid = E_PAD // B_E
    return pl.pallas_call(
        _edge_filter_body,
        grid=(grid,),
        in_specs=[
            pl.BlockSpec((B_E, 32), lambda i: (i, 0)),
            pl.BlockSpec((B_E, 1), lambda i: (i, 0)),
            pl.BlockSpec((32, NF), lambda i: (0, 0)),
            pl.BlockSpec((2, NF, FH), lambda i: (0, 0, 0)),
            pl.BlockSpec((1, NF), lambda i: (0, 0)),
            pl.BlockSpec((2, 1, FH), lambda i: (0, 0, 0)),
        ],
        out_specs=pl.BlockSpec((2, B_E, FH), lambda i: (0, i, 0)),
        out_shape=jax.ShapeDtypeStruct((2, E_PAD, FH), jnp.float32),
        interpret=_INTERP,
    )(fij, cij, wf1p, wf2h, bf1, bf2h)


def _init_body(z_ref, emb_ref, winh_ref, feat_ref, h_ref):
    z = z_ref[...]
    lane = jax.lax.broadcasted_iota(jnp.int32, (1, NF), 1)
    oh = (z == lane).astype(jnp.float32)
    feat = jnp.dot(oh, emb_ref[...], preferred_element_type=jnp.float32)
    feat_ref[...] = feat
    for half in range(2):
        h_ref[half] = jnp.dot(feat, winh_ref[half],
                              preferred_element_type=jnp.float32)


def _init_feat(z_col, emb_pad, winh0):
    grid = N // B_N
    return pl.pallas_call(
        _init_body,
        grid=(grid,),
        in_specs=[
            pl.BlockSpec((B_N, 1), lambda i: (i, 0)),
            pl.BlockSpec((NF, NF), lambda i: (0, 0)),
            pl.BlockSpec((2, NF, FH), lambda i: (0, 0, 0)),
        ],
        out_specs=[
            pl.BlockSpec((B_N, NF), lambda i: (i, 0)),
            pl.BlockSpec((2, B_N, FH), lambda i: (0, i, 0)),
        ],
        out_shape=[
            jax.ShapeDtypeStruct((N, NF), jnp.float32),
            jax.ShapeDtypeStruct((2, N, FH), jnp.float32),
        ],
        interpret=_INTERP,
    )(z_col, emb_pad, winh0)


def _node_body(feat_ref, m_ref, wout1_ref, wout2_ref, bout1_ref, bout2_ref,
               gamma_ref, beta_ref, winh_next_ref, feat_out_ref, h_out_ref):
    m = jnp.concatenate([m_ref[0], m_ref[1]], axis=1)
    u = jnp.dot(m, wout1_ref[...], preferred_element_type=jnp.float32) + bout1_ref[...]
    su = u * jax.nn.sigmoid(u)
    mm = jnp.dot(su, wout2_ref[...], preferred_element_type=jnp.float32) + bout2_ref[...]
    f = feat_ref[...] + mm
    mu = jnp.mean(f, axis=1, keepdims=True)
    d = f - mu
    var = jnp.mean(d * d, axis=1, keepdims=True)
    fn = gamma_ref[...] * d * jax.lax.rsqrt(var + 1e-5) + beta_ref[...]
    feat_out_ref[...] = fn
    for half in range(2):
        h_out_ref[half] = jnp.dot(fn, winh_next_ref[half],
                                  preferred_element_type=jnp.float32)


def _node_update(feat, m2, wout1, wout2, bout1, bout2, gamma, beta, winh_next):
    grid = N // B_N
    return pl.pallas_call(
        _node_body,
        grid=(grid,),
        in_specs=[
            pl.BlockSpec((B_N, NF), lambda i: (i, 0)),
            pl.BlockSpec((2, B_N, FH), lambda i: (0, i, 0)),
            pl.BlockSpec((NF, NF), lambda i: (0, 0)),
            pl.BlockSpec((NF, NF), lambda i: (0, 0)),
            pl.BlockSpec((1, NF), lambda i: (0, 0)),
            pl.BlockSpec((1, NF), lambda i: (0, 0)),
            pl.BlockSpec((1, NF), lambda i: (0, 0)),
            pl.BlockSpec((1, NF), lambda i: (0, 0)),
            pl.BlockSpec((2, NF, FH), lambda i: (0, 0, 0)),
        ],
        out_specs=[
            pl.BlockSpec((B_N, NF), lambda i: (i, 0)),
            pl.BlockSpec((2, B_N, FH), lambda i: (0, i, 0)),
        ],
        out_shape=[
            jax.ShapeDtypeStruct((N, NF), jnp.float32),
            jax.ShapeDtypeStruct((2, N, FH), jnp.float32),
        ],
        interpret=_INTERP,
    )(feat, m2, wout1, wout2, bout1, bout2, gamma, beta, winh_next)


def _sc_mesh():
    return plsc.VectorSubcoreMesh(core_axis_name="c", subcore_axis_name="s",
                                  num_cores=NC, num_subcores=NS)


def _sc_d2_body(px_hbm, py_hbm, pz_hbm, src_hbm, dst_hbm, d2_hbm,
                px_v, py_v, pz_v, src_v, dst_v, d2_v):
    cid = lax.axis_index("c")
    sid = lax.axis_index("s")
    wid = sid * NC + cid
    pltpu.sync_copy(px_hbm, px_v)
    pltpu.sync_copy(py_hbm, py_v)
    pltpu.sync_copy(pz_hbm, pz_v)
    lanes = jnp.arange(16, dtype=jnp.int32)
    wbase = wid * EPW

    @pl.loop(0, EPW // CE_D2)
    def _chunk(ci):
        base = wbase + ci * CE_D2
        pltpu.sync_copy(src_hbm.at[pl.ds(base, CE_D2)], src_v)
        pltpu.sync_copy(dst_hbm.at[pl.ds(base, CE_D2)], dst_v)

        @pl.loop(0, CE_D2 // 16)
        def _vec(j):
            sl = pl.ds(j * 16, 16)
            si = src_v[sl]
            di = dst_v[sl]
            dx = plsc.load_gather(px_v, [di]) - plsc.load_gather(px_v, [si])
            dy = plsc.load_gather(py_v, [di]) - plsc.load_gather(py_v, [si])
            dz = plsc.load_gather(pz_v, [di]) - plsc.load_gather(pz_v, [si])
            d2 = dx * dx + dy * dy + dz * dz
            eid = base + j * 16 + lanes
            d2_v[sl] = jnp.where(eid < E, d2, 100.0)

        pltpu.sync_copy(d2_v, d2_hbm.at[pl.ds(base, CE_D2)])


def _sc_d2(px, py, pz, src, dst):
    f = pl.kernel(
        _sc_d2_body,
        out_type=jax.ShapeDtypeStruct((E_PAD,), jnp.float32),
        mesh=_sc_mesh(),
        compiler_params=pltpu.CompilerParams(needs_layout_passes=False),
        scratch_types=[
            pltpu.VMEM((N_PADR,), jnp.float32),
            pltpu.VMEM((N_PADR,), jnp.float32),
            pltpu.VMEM((N_PADR,), jnp.float32),
            pltpu.VMEM((CE_D2,), jnp.int32),
            pltpu.VMEM((CE_D2,), jnp.int32),
            pltpu.VMEM((CE_D2,), jnp.float32),
        ],
    )
    return f(px, py, pz, src, dst)


def _sc_cfconv_body(h_hbm, wc_hbm, src_hbm, dst_hbm, out_hbm,
                    src_blk, dst_blk, hrows0, hrows1, wcv0, wcv1, sbuf0, sbuf1,
                    macc, gsem0, gsem1, wsem0, wsem1, ssem0, ssem1):
    cid = lax.axis_index("c")
    sid = lax.axis_index("s")
    hrows = (hrows0, hrows1)
    wcv = (wcv0, wcv1)
    sbuf = (sbuf0, sbuf1)
    gsem = (gsem0, gsem1)
    wsem = (wsem0, wsem1)
    ssem = (ssem0, ssem1)
    NCH = EPT // CE

    @pl.loop(0, ZROWS)
    def _zr(i):
        for k2 in range(FH // 16):
            hrows0[i, pl.ds(k2 * 16, 16)] = jnp.zeros((16,), jnp.float32)

    row0 = sid * ROWS_PER_TILE
    for k in range(ROWS_PER_TILE // ZROWS):
        pltpu.sync_copy(hrows0.at[pl.ds(0, ZROWS)],
                        macc.at[pl.ds(row0 + k * ZROWS, ZROWS)])
    ztail = ROWS_PER_TILE % ZROWS
    if ztail:
        pltpu.sync_copy(hrows0.at[pl.ds(0, ztail)],
                        macc.at[pl.ds(row0 + ROWS_PER_TILE - ztail, ztail)])
    plsc.subcore_barrier()

    pltpu.sync_copy(src_hbm.at[pl.ds(sid * NCH, NCH)], src_blk)
    pltpu.sync_copy(dst_hbm.at[pl.ds(sid * NCH, NCH)], dst_blk)

    def start_gather(ci, b):
        pltpu.async_copy(h_hbm.at[cid].at[src_blk.at[ci]], hrows[b], gsem[b])

    def start_wc(ci, b):
        base = sid * EPT + ci * CE
        pltpu.async_copy(wc_hbm.at[cid, pl.ds(base, CE)], wcv[b], wsem[b])

    start_gather(0, 0)
    start_gather(1, 1)
    start_wc(0, 0)
    start_wc(1, 1)

    @pl.loop(0, NCH, step=2)
    def _chunk(ci0):
        for b in range(2):
            ci = ci0 + b
            base = sid * EPT + ci * CE
            pltpu.make_async_copy(h_hbm.at[cid].at[src_blk.at[ci]], hrows[b],
                                  gsem[b]).wait()
            pltpu.make_async_copy(wc_hbm.at[cid, pl.ds(base, CE)], wcv[b],
                                  wsem[b]).wait()

            @pl.when(ci >= 2)
            def _():
                pltpu.make_async_copy(sbuf[b], macc.at[dst_blk.at[ci - 2]],
                                      ssem[b]).wait()

            @pl.loop(0, CE)
            def _row(i):
                for k2 in range(FH // 16):
                    sl = pl.ds(k2 * 16, 16)
                    sbuf[b][i, sl] = hrows[b][i, sl] * wcv[b][i, sl]

            pltpu.async_copy(sbuf[b], macc.at[dst_blk.at[ci]], ssem[b],
                             add=True)

            @pl.when(ci + 2 < NCH)
            def _():
                start_gather(ci + 2, b)
                start_wc(ci + 2, b)

    for b in range(2):
        pltpu.make_async_copy(sbuf[b], macc.at[dst_blk.at[NCH - 2 + b]],
                              ssem[b]).wait()

    plsc.subcore_barrier()
    pltpu.sync_copy(macc.at[pl.ds(row0, ROWS_PER_TILE)],
                    out_hbm.at[cid, pl.ds(row0, ROWS_PER_TILE)])


def _sc_cfconv(h2, wc2, src2, dst2):
    f = pl.kernel(
        _sc_cfconv_body,
        out_type=jax.ShapeDtypeStruct((NC, NACC, FH), jnp.float32),
        mesh=_sc_mesh(),
        compiler_params=pltpu.CompilerParams(needs_layout_passes=False,
                                             use_tc_tiling_on_sc=False),
        scratch_types=(
            [pltpu.VMEM((EPT // CE, CE), jnp.int32) for _ in range(2)]
            + [pltpu.VMEM((CE, FH), jnp.float32) for _ in range(6)]
            + [pltpu.VMEM_SHARED((NACC, FH), jnp.float32)]
            + [pltpu.SemaphoreType.DMA for _ in range(6)]
        ),
    )
    return f(h2, wc2, src2, dst2)


def kernel(z, pos, edge_index, emb, Wf1, bf1, Wf2, bf2, Win, Wout1, bout1, Wout2, bout2, gamma, beta):
    src = jnp.pad(edge_index[0].astype(jnp.int32), (0, E_PAD - E))
    dst = jnp.pad(edge_index[1].astype(jnp.int32), (0, E_PAD - E))

    pos_t = jnp.pad(pos.T, ((0, 0), (0, N_PADR - N)))
    d2 = _sc_d2(pos_t[0], pos_t[1], pos_t[2], src, dst)
    d2_col = d2.reshape(E_PAD, 1)

    wf1p = jnp.pad(Wf1, ((0, 0), (0, 32 - NB), (0, 0))).astype(jnp.bfloat16)
    emb_pad = jnp.pad(emb, ((0, NF - emb.shape[0]), (0, 0)))
    wf2h = Wf2.reshape(N_INTER, NF, 2, FH).transpose(0, 2, 1, 3).astype(jnp.bfloat16)
    winh = Win.reshape(N_INTER, NF, 2, FH).transpose(0, 2, 1, 3)
    bf2h = bf2.reshape(N_INTER, 2, 1, FH)
    b2 = lambda b: b.reshape(N_INTER, 1, NF)
    bf1c, bout1c, bout2c = b2(bf1), b2(bout1), b2(bout2)
    gammac, betac = b2(gamma), b2(beta)

    src2 = src.reshape(E_PAD // CE, CE)
    dst2 = dst.reshape(E_PAD // CE, CE)
    z_col = z.reshape(N, 1).astype(jnp.int32)

    fij, cij = _smear(d2_col)
    wcs = [_edge_filter(fij, cij, wf1p[i], wf2h[i], bf1c[i], bf2h[i])
           for i in range(N_INTER)]
    feat, h2 = _init_feat(z_col, emb_pad, winh[0])

    for i in range(N_INTER):
        m2 = _sc_cfconv(h2, wcs[i], src2, dst2)
        feat, h2 = _node_update(feat, m2[:, :N], Wout1[i], Wout2[i],
                                bout1c[i], bout2c[i], gammac[i], betac[i],
                                winh[(i + 1) % N_INTER])
    return feat
